# baseline (speedup 1.0000x reference)
import functools

import jax
import jax.numpy as jnp
from jax import lax
from jax.experimental import pallas as pl
from jax.experimental.pallas import tpu as pltpu
from jax.experimental.pallas import tpu_sc as plsc

B, T, H = 1024, 50, 64
N_VOCAB = 1000000
N = B * T
NC, NS = 2, 16
NW = NC * NS
RPW = N // NW
NQ = 4
QR = RPW // NQ
CH = 80
NCH = QR // CH
HP = 2 * H
LANES = 16

_mesh = plsc.VectorSubcoreMesh(core_axis_name="c", subcore_axis_name="s",
                               num_cores=NC, num_subcores=NS)

TBLK = 512


def _tbody(emb_t_ref, out_ref):
    xt = jnp.swapaxes(emb_t_ref[...], 0, 1)
    out_ref[...] = jnp.concatenate(
        [xt, jnp.zeros((TBLK, HP - H), jnp.float32)], axis=1)


def _transpose_pad(emb_t):
    grid = (N_VOCAB + TBLK - 1) // TBLK
    return pl.pallas_call(
        _tbody,
        grid=(grid,),
        in_specs=[pl.BlockSpec((H, TBLK), lambda i: (0, i))],
        out_specs=pl.BlockSpec((TBLK, HP), lambda i: (i, 0)),
        out_shape=jax.ShapeDtypeStruct((N_VOCAB, HP), jnp.float32),
    )(emb_t)


@functools.partial(
    pl.kernel,
    out_type=(
        jax.ShapeDtypeStruct((N, H), jnp.float32),
        jax.ShapeDtypeStruct((N,), jnp.float32),
    ),
    mesh=_mesh,
    compiler_params=pltpu.CompilerParams(use_tc_tiling_on_sc=False,
                                         needs_layout_passes=False),
    scratch_types=[
        pltpu.VMEM((RPW,), jnp.int32),
        pltpu.VMEM((RPW,), jnp.int32),
        pltpu.VMEM((B,), jnp.int32),
        pltpu.VMEM((RPW,), jnp.float32),
        pltpu.VMEM((QR, HP), jnp.float32),
        pltpu.VMEM((QR, HP), jnp.float32),
        pltpu.VMEM((160, H), jnp.float32),
        pltpu.SemaphoreType.DMA,
        pltpu.SemaphoreType.DMA,
    ],
)
def _encode(x_hbm, cidx_hbm, xlen_hbm, emb_hbm, ctab_hbm,
            out_hbm, mask_hbm,
            idx_v, cidx_v, xlen_v, mask_v, erows_v, erows_v2, ctab_v, sem, sem2):
    wid = lax.axis_index("s") * NC + lax.axis_index("c")
    base = wid * RPW

    pltpu.sync_copy(x_hbm.at[pl.ds(base, RPW)], idx_v)
    waits = []
    for j in range(NCH):
        waits.append(pltpu.async_copy(
            emb_hbm.at[idx_v.at[pl.ds(j * CH, CH)]],
            erows_v.at[pl.ds(j * CH, CH)], sem))
    pltpu.sync_copy(cidx_hbm.at[pl.ds(base, RPW)], cidx_v)
    pltpu.sync_copy(xlen_hbm, xlen_v)
    pltpu.sync_copy(ctab_hbm, ctab_v)

    t_vec = jnp.full((LANES,), T, jnp.int32)
    ones = jnp.full((LANES,), 1.0, jnp.float32)
    zeros = jnp.full((LANES,), 0.0, jnp.float32)

    @plsc.parallel_loop(0, RPW // LANES, step=1, unroll=8)
    def mask_body(i):
        n = lax.iota(jnp.int32, LANES) + (base + i * LANES)
        b = lax.div(n, t_vec)
        t = n - b * T
        lens = plsc.load_gather(xlen_v, [b])
        mask_v[pl.ds(i * LANES, LANES)] = jnp.where(t < lens, ones, zeros)

    bufs = (erows_v, erows_v2)
    for q in range(NQ):
        qbase = q * QR
        for w in waits:
            w.wait()
        waits = []
        if q + 1 < NQ:
            nxt = bufs[(q + 1) % 2]
            for j in range(NCH):
                waits.append(pltpu.async_copy(
                    emb_hbm.at[idx_v.at[pl.ds(qbase + QR + j * CH, CH)]],
                    nxt.at[pl.ds(j * CH, CH)], sem))

        buf = bufs[q % 2]

        @plsc.parallel_loop(0, QR, step=1, unroll=4)
        def row_body(r):
            rb = jnp.full((LANES,), qbase, jnp.int32) + r
            m16 = plsc.load_gather(mask_v, [rb])
            cvec = plsc.load_gather(cidx_v, [rb])
            for c in range(H // LANES):
                sl = pl.ds(c * LANES, LANES)
                col = lax.iota(jnp.int32, LANES) + (c * LANES)
                cval = plsc.load_gather(ctab_v, [cvec, col])
                buf[r, sl] = (buf[r, sl] + cval) * m16

        pltpu.sync_copy(buf.at[:, pl.ds(0, H)],
                        out_hbm.at[pl.ds(base + qbase, QR)])

    pltpu.sync_copy(mask_v, mask_hbm.at[pl.ds(base, RPW)])


def kernel(x, x_lengths, tone, language, ja_bert, noise_scale, emb_w,
           tone_w, lang_w):
    del ja_bert
    x_f = x.reshape(N).astype(jnp.int32)
    cidx = (tone.astype(jnp.int32) * 10 + language.astype(jnp.int32)).reshape(N)
    xlen = x_lengths.astype(jnp.int32)
    ctab = (tone_w.astype(jnp.float32)[:, None, :]
            + lang_w.astype(jnp.float32)[None, :, :]).reshape(160, H)

    embp = _transpose_pad(emb_w.T)
    out_flat, mask_flat = _encode(x_f, cidx, xlen, embp, ctab)

    out = out_flat.reshape(B, T, H)
    x_mask = mask_flat.reshape(B, 1, T)
    z = jax.random.normal(jax.random.key(1234), (B, 2, T),
                          jnp.float32) * noise_scale
    z0, z1 = z[:, 0:1, :], z[:, 1:2, :]
    return (out, x_mask, z0, z1)


# speedup vs baseline: 2.1049x; 2.1049x over previous
import functools

import jax
import jax.numpy as jnp
from jax import lax
from jax.experimental import pallas as pl
from jax.experimental.pallas import tpu as pltpu
from jax.experimental.pallas import tpu_sc as plsc

B, T, H = 1024, 50, 64
N = B * T
NC, NS = 2, 16
NW = NC * NS
RPW = N // NW
NQ = 4
QR = RPW // NQ
CH = 80
NCH = QR // CH
HP = 2 * H
LANES = 16

_mesh = plsc.VectorSubcoreMesh(core_axis_name="c", subcore_axis_name="s",
                               num_cores=NC, num_subcores=NS)


@functools.partial(
    pl.kernel,
    out_type=(
        jax.ShapeDtypeStruct((N, H), jnp.float32),
        jax.ShapeDtypeStruct((N,), jnp.float32),
    ),
    mesh=_mesh,
    compiler_params=pltpu.CompilerParams(use_tc_tiling_on_sc=False,
                                         needs_layout_passes=False),
    scratch_types=[
        pltpu.VMEM((RPW,), jnp.int32),
        pltpu.VMEM((RPW,), jnp.int32),
        pltpu.VMEM((B,), jnp.int32),
        pltpu.VMEM((RPW,), jnp.float32),
        pltpu.VMEM((QR, HP), jnp.float32),
        pltpu.VMEM((QR, HP), jnp.float32),
        pltpu.VMEM((160, H), jnp.float32),
        pltpu.SemaphoreType.DMA,
        pltpu.SemaphoreType.DMA,
    ],
)
def _encode(x_hbm, cidx_hbm, xlen_hbm, emb_hbm, ctab_hbm,
            out_hbm, mask_hbm,
            idx_v, cidx_v, xlen_v, mask_v, erows_v, erows_v2, ctab_v, sem, sem2):
    wid = lax.axis_index("s") * NC + lax.axis_index("c")
    base = wid * RPW

    pltpu.sync_copy(x_hbm.at[pl.ds(base, RPW)], idx_v)
    waits = []
    for j in range(NCH):
        waits.append(pltpu.async_copy(
            emb_hbm.at[idx_v.at[pl.ds(j * CH, CH)]],
            erows_v.at[pl.ds(j * CH, CH)], sem))
    pltpu.sync_copy(cidx_hbm.at[pl.ds(base, RPW)], cidx_v)
    pltpu.sync_copy(xlen_hbm, xlen_v)
    pltpu.sync_copy(ctab_hbm, ctab_v)

    t_vec = jnp.full((LANES,), T, jnp.int32)
    ones = jnp.full((LANES,), 1.0, jnp.float32)
    zeros = jnp.full((LANES,), 0.0, jnp.float32)

    @plsc.parallel_loop(0, RPW // LANES, step=1, unroll=8)
    def mask_body(i):
        n = lax.iota(jnp.int32, LANES) + (base + i * LANES)
        b = lax.div(n, t_vec)
        t = n - b * T
        lens = plsc.load_gather(xlen_v, [b])
        mask_v[pl.ds(i * LANES, LANES)] = jnp.where(t < lens, ones, zeros)

    bufs = (erows_v, erows_v2)
    for q in range(NQ):
        qbase = q * QR
        for w in waits:
            w.wait()
        waits = []
        if q + 1 < NQ:
            nxt = bufs[(q + 1) % 2]
            for j in range(NCH):
                waits.append(pltpu.async_copy(
                    emb_hbm.at[idx_v.at[pl.ds(qbase + QR + j * CH, CH)]],
                    nxt.at[pl.ds(j * CH, CH)], sem))

        buf = bufs[q % 2]

        @plsc.parallel_loop(0, QR, step=1, unroll=4)
        def row_body(r):
            rb = jnp.full((LANES,), qbase, jnp.int32) + r
            m16 = plsc.load_gather(mask_v, [rb])
            cvec = plsc.load_gather(cidx_v, [rb])
            for c in range(H // LANES):
                sl = pl.ds(c * LANES, LANES)
                col = lax.iota(jnp.int32, LANES) + (c * LANES)
                cval = plsc.load_gather(ctab_v, [cvec, col])
                buf[r, sl] = (buf[r, sl] + cval) * m16

        pltpu.sync_copy(buf.at[:, pl.ds(0, H)],
                        out_hbm.at[pl.ds(base + qbase, QR)])

    pltpu.sync_copy(mask_v, mask_hbm.at[pl.ds(base, RPW)])


def kernel(x, x_lengths, tone, language, ja_bert, noise_scale, emb_w,
           tone_w, lang_w):
    del ja_bert
    x_f = x.reshape(N).astype(jnp.int32)
    cidx = (tone.astype(jnp.int32) * 10 + language.astype(jnp.int32)).reshape(N)
    xlen = x_lengths.astype(jnp.int32)
    ctab = (tone_w.astype(jnp.float32)[:, None, :]
            + lang_w.astype(jnp.float32)[None, :, :]).reshape(160, H)

    embp = jnp.pad(emb_w, ((0, 0), (0, H)))
    out_flat, mask_flat = _encode(x_f, cidx, xlen, embp, ctab)

    out = out_flat.reshape(B, T, H)
    x_mask = mask_flat.reshape(B, 1, T)
    z = jax.random.normal(jax.random.key(1234), (B, 2, T),
                          jnp.float32) * noise_scale
    z0, z1 = z[:, 0:1, :], z[:, 1:2, :]
    return (out, x_mask, z0, z1)


# speedup vs baseline: 2.1059x; 1.0005x over previous
import functools

import jax
import jax.numpy as jnp
from jax import lax
from jax.experimental import pallas as pl
from jax.experimental.pallas import tpu as pltpu
from jax.experimental.pallas import tpu_sc as plsc

B, T, H = 1024, 50, 64
N = B * T
NC, NS = 2, 16
NW = NC * NS
RPW = N // NW
NQ = 4
QR = RPW // NQ
CH = 80
NCH = QR // CH
HP = 2 * H
LANES = 16

_mesh = plsc.VectorSubcoreMesh(core_axis_name="c", subcore_axis_name="s",
                               num_cores=NC, num_subcores=NS)


@functools.partial(
    pl.kernel,
    out_type=(
        jax.ShapeDtypeStruct((N, H), jnp.float32),
        jax.ShapeDtypeStruct((N,), jnp.float32),
    ),
    mesh=_mesh,
    compiler_params=pltpu.CompilerParams(use_tc_tiling_on_sc=False,
                                         needs_layout_passes=False),
    scratch_types=[
        pltpu.VMEM((RPW,), jnp.int32),
        pltpu.VMEM((RPW,), jnp.int32),
        pltpu.VMEM((B,), jnp.int32),
        pltpu.VMEM((RPW,), jnp.float32),
        pltpu.VMEM((QR, HP), jnp.float32),
        pltpu.VMEM((QR, HP), jnp.float32),
        pltpu.VMEM((160, H), jnp.float32),
        pltpu.SemaphoreType.DMA,
        pltpu.SemaphoreType.DMA,
    ],
)
def _encode(x_hbm, cidx_hbm, xlen_hbm, emb_hbm, ctab_hbm,
            out_hbm, mask_hbm,
            idx_v, cidx_v, xlen_v, mask_v, erows_v, erows_v2, ctab_v, sem, sem2):
    wid = lax.axis_index("s") * NC + lax.axis_index("c")
    base = wid * RPW

    pltpu.sync_copy(x_hbm.at[pl.ds(base, RPW)], idx_v)
    waits = []
    for j in range(NCH):
        waits.append(pltpu.async_copy(
            emb_hbm.at[idx_v.at[pl.ds(j * CH, CH)]],
            erows_v.at[pl.ds(j * CH, CH)], sem))
    pltpu.sync_copy(cidx_hbm.at[pl.ds(base, RPW)], cidx_v)
    pltpu.sync_copy(xlen_hbm, xlen_v)
    pltpu.sync_copy(ctab_hbm, ctab_v)

    t_vec = jnp.full((LANES,), T, jnp.int32)
    ones = jnp.full((LANES,), 1.0, jnp.float32)
    zeros = jnp.full((LANES,), 0.0, jnp.float32)

    @plsc.parallel_loop(0, RPW // LANES, step=1, unroll=8)
    def mask_body(i):
        n = lax.iota(jnp.int32, LANES) + (base + i * LANES)
        b = lax.div(n, t_vec)
        t = n - b * T
        lens = plsc.load_gather(xlen_v, [b])
        mask_v[pl.ds(i * LANES, LANES)] = jnp.where(t < lens, ones, zeros)

    bufs = (erows_v, erows_v2)
    for q in range(NQ):
        qbase = q * QR
        for w in waits:
            w.wait()
        waits = []
        if q + 1 < NQ:
            nxt = bufs[(q + 1) % 2]
            for j in range(NCH):
                waits.append(pltpu.async_copy(
                    emb_hbm.at[idx_v.at[pl.ds(qbase + QR + j * CH, CH)]],
                    nxt.at[pl.ds(j * CH, CH)], sem))

        buf = bufs[q % 2]

        @plsc.parallel_loop(0, QR, step=1, unroll=4)
        def row_body(r):
            rb = jnp.full((LANES,), qbase, jnp.int32) + r
            m16 = plsc.load_gather(mask_v, [rb])
            cvec = plsc.load_gather(cidx_v, [rb])
            for c in range(H // LANES):
                sl = pl.ds(c * LANES, LANES)
                col = lax.iota(jnp.int32, LANES) + (c * LANES)
                cval = plsc.load_gather(ctab_v, [cvec, col])
                buf[r, sl] = (buf[r, sl] + cval) * m16

        pltpu.sync_copy(buf.at[:, pl.ds(0, H)],
                        out_hbm.at[pl.ds(base + qbase, QR)])

    pltpu.sync_copy(mask_v, mask_hbm.at[pl.ds(base, RPW)])


def kernel(x, x_lengths, tone, language, ja_bert, noise_scale, emb_w,
           tone_w, lang_w):
    del ja_bert
    x_f = x.reshape(N).astype(jnp.int32)
    cidx = (tone.astype(jnp.int32) * 10 + language.astype(jnp.int32)).reshape(N)
    xlen = x_lengths.astype(jnp.int32)
    ctab = (tone_w.astype(jnp.float32)[:, None, :]
            + lang_w.astype(jnp.float32)[None, :, :]).reshape(160, H)

    embp = jnp.pad(emb_w.T, ((0, H), (0, 0))).T
    out_flat, mask_flat = _encode(x_f, cidx, xlen, embp, ctab)

    out = out_flat.reshape(B, T, H)
    x_mask = mask_flat.reshape(B, 1, T)
    z = jax.random.normal(jax.random.key(1234), (B, 2, T),
                          jnp.float32) * noise_scale
    z0, z1 = z[:, 0:1, :], z[:, 1:2, :]
    return (out, x_mask, z0, z1)


# speedup vs baseline: 2.1064x; 1.0002x over previous
import functools

import jax
import jax.numpy as jnp
from jax import lax
from jax.experimental import pallas as pl
from jax.experimental.pallas import tpu as pltpu
from jax.experimental.pallas import tpu_sc as plsc

B, T, H = 1024, 50, 64
N = B * T
NC, NS = 2, 16
NW = NC * NS
RPW = N // NW
NQ = 4
QR = RPW // NQ
CH = 80
NCH = QR // CH
HP = 2 * H
LANES = 16

_mesh = plsc.VectorSubcoreMesh(core_axis_name="c", subcore_axis_name="s",
                               num_cores=NC, num_subcores=NS)


@functools.partial(
    pl.kernel,
    out_type=(
        jax.ShapeDtypeStruct((N, H), jnp.float32),
        jax.ShapeDtypeStruct((N,), jnp.float32),
    ),
    mesh=_mesh,
    compiler_params=pltpu.CompilerParams(use_tc_tiling_on_sc=False,
                                         needs_layout_passes=False),
    scratch_types=[
        pltpu.VMEM((RPW,), jnp.int32),
        pltpu.VMEM((RPW,), jnp.int32),
        pltpu.VMEM((B,), jnp.int32),
        pltpu.VMEM((RPW,), jnp.float32),
        pltpu.VMEM((QR, HP), jnp.float32),
        pltpu.VMEM((QR, HP), jnp.float32),
        pltpu.VMEM((160, H), jnp.float32),
        pltpu.SemaphoreType.DMA,
        pltpu.SemaphoreType.DMA,
    ],
)
def _encode(x_hbm, cidx_hbm, xlen_hbm, emb_hbm, ctab_hbm,
            out_hbm, mask_hbm,
            idx_v, cidx_v, xlen_v, mask_v, erows_v, erows_v2, ctab_v, sem, sem2):
    wid = lax.axis_index("s") * NC + lax.axis_index("c")
    base = wid * RPW

    pltpu.sync_copy(x_hbm.at[pl.ds(base, RPW)], idx_v)
    waits = []
    for j in range(NCH):
        waits.append(pltpu.async_copy(
            emb_hbm.at[idx_v.at[pl.ds(j * CH, CH)]],
            erows_v.at[pl.ds(j * CH, CH)], sem))
    pltpu.sync_copy(cidx_hbm.at[pl.ds(base, RPW)], cidx_v)
    pltpu.sync_copy(xlen_hbm, xlen_v)
    pltpu.sync_copy(ctab_hbm, ctab_v)

    t_vec = jnp.full((LANES,), T, jnp.int32)
    ones = jnp.full((LANES,), 1.0, jnp.float32)
    zeros = jnp.full((LANES,), 0.0, jnp.float32)

    @plsc.parallel_loop(0, RPW // LANES, step=1, unroll=8)
    def mask_body(i):
        n = lax.iota(jnp.int32, LANES) + (base + i * LANES)
        b = lax.div(n, t_vec)
        t = n - b * T
        lens = plsc.load_gather(xlen_v, [b])
        mask_v[pl.ds(i * LANES, LANES)] = jnp.where(t < lens, ones, zeros)

    bufs = (erows_v, erows_v2)
    for q in range(NQ):
        qbase = q * QR
        for w in waits:
            w.wait()
        waits = []
        if q + 1 < NQ:
            nxt = bufs[(q + 1) % 2]
            for j in range(NCH):
                waits.append(pltpu.async_copy(
                    emb_hbm.at[idx_v.at[pl.ds(qbase + QR + j * CH, CH)]],
                    nxt.at[pl.ds(j * CH, CH)], sem))

        buf = bufs[q % 2]

        @plsc.parallel_loop(0, QR, step=1, unroll=4)
        def row_body(r):
            rb = jnp.full((LANES,), qbase, jnp.int32) + r
            m16 = plsc.load_gather(mask_v, [rb])
            cvec = plsc.load_gather(cidx_v, [rb])
            for c in range(H // LANES):
                sl = pl.ds(c * LANES, LANES)
                col = lax.iota(jnp.int32, LANES) + (c * LANES)
                cval = plsc.load_gather(ctab_v, [cvec, col])
                buf[r, sl] = (buf[r, sl] + cval) * m16

        pltpu.sync_copy(buf.at[:, pl.ds(0, H)],
                        out_hbm.at[pl.ds(base + qbase, QR)])

    pltpu.sync_copy(mask_v, mask_hbm.at[pl.ds(base, RPW)])


def kernel(x, x_lengths, tone, language, ja_bert, noise_scale, emb_w,
           tone_w, lang_w):
    del ja_bert
    x_f = x.reshape(N).astype(jnp.int32)
    cidx = (tone.astype(jnp.int32) * 10 + language.astype(jnp.int32)).reshape(N)
    xlen = x_lengths.astype(jnp.int32)
    ctab = (tone_w.astype(jnp.float32)[:, None, :]
            + lang_w.astype(jnp.float32)[None, :, :]).reshape(160, H)

    embp = jnp.pad(emb_w, ((0, 0), (0, H)))
    out_flat, mask_flat = _encode(x_f, cidx, xlen, embp, ctab)

    out = out_flat.reshape(B, T, H)
    x_mask = mask_flat.reshape(B, 1, T)
    z = jax.random.normal(jax.random.key(1234), (B, 2, T),
                          jnp.float32) * noise_scale
    z0, z1 = z[:, 0:1, :], z[:, 1:2, :]
    return (out, x_mask, z0, z1)
